# baseline (speedup 1.0000x reference)
import jax
import jax.numpy as jnp
from jax import lax
from jax.experimental import pallas as pl
from jax.experimental.pallas import tpu as pltpu
from jax.experimental.pallas import tpu_sc as plsc

_MIN_N, _MAX_N = 2, 4
_NUM_HEADS = 4
_TABLE_SIZE = 100000
_EMBED_DIM = 64
_B, _S = 1024, 50
_BS = _B * _S


def _get_prime(n):
    def is_prime(x):
        if x < 2:
            return False
        for i in range(2, int(x ** 0.5) + 1):
            if x % i == 0:
                return False
        return True
    while not is_prime(n):
        n += 1
    return n


_BASES = [_get_prime(i * 100 + 31) for i in range(_NUM_HEADS)]
_MODULI = [_get_prime(_TABLE_SIZE + i * 1000) for i in range(_NUM_HEADS)]

_C0 = {n: [[pow(_BASES[k], i, _MODULI[k]) for i in range(n)]
           for k in range(_NUM_HEADS)] for n in range(_MIN_N, _MAX_N + 1)}
_C1 = {n: [[(256 * pow(_BASES[k], i, _MODULI[k])) % _MODULI[k] for i in range(n)]
           for k in range(_NUM_HEADS)] for n in range(_MIN_N, _MAX_N + 1)}
_ROW0 = {2: 0, 3: 2, 4: 5}

_NC, _NS = 2, 16
_NW = _NC * _NS
_C = _BS // _NW
_CH = 80
_NCH = _C // _CH
_NJ = (_MAX_N - _MIN_N + 1) * _NUM_HEADS

_TB = 800
_TASKS_PER_TAB = _TABLE_SIZE // _TB
_NTASK = _NJ * _TASKS_PER_TAB
_MSTEPS = -(-_NTASK // _NW)
_NDB = _EMBED_DIM // 16


def _body(ng_hbm, s2_hbm, s3_hbm, s4_hbm, out_hbm,
          rm_hbm, flags_hbm,
          ng_v, idx_v, tin0_v, tin1_v, tout_v, buf0_v, buf1_v,
          flg_v, sem0, sem1, sem2, sem3):
    i32 = jnp.int32
    w = lax.axis_index("s") * i32(_NC) + lax.axis_index("c")
    base_row = w * i32(_C)
    iota16 = lax.iota(jnp.int32, 16)

    flg_v[i32(0), :] = iota16 * i32(0)
    pltpu.sync_copy(flg_v.at[i32(0)], flags_hbm.at[w])

    pltpu.sync_copy(ng_hbm.at[w], ng_v)

    def hstep(c, carry):
        for t5 in range(_CH // 16):
            off = c * i32(_CH) + i32(t5 * 16)
            for n in range(_MIN_N, _MAX_N + 1):
                r0 = _ROW0[n]
                gs = [ng_v[r0 + i, pl.ds(off, 16)] for i in range(n)]
                g1 = [g >> 8 for g in gs]
                g0 = [g & 255 for g in gs]
                for k in range(_NUM_HEADS):
                    j = (n - _MIN_N) * _NUM_HEADS + k
                    acc = g1[0] * _C1[n][k][0] + g0[0] * _C0[n][k][0]
                    for i in range(1, n):
                        acc = acc + g1[i] * _C1[n][k][i] + g0[i] * _C0[n][k][i]
                    h = acc % _MODULI[k]
                    h = jnp.where(h >= _TABLE_SIZE, h - _TABLE_SIZE, h)
                    idx_v[j, c, pl.ds(t5 * 16, 16)] = h + j * _TABLE_SIZE
        return carry

    lax.fori_loop(jnp.int32(0), jnp.int32(_NCH), hstep, jnp.int32(0))

    srcs = (s2_hbm, s3_hbm, s4_hbm)
    tins = (tin0_v, tin1_v)
    tsems = (sem0, sem1)

    def t_in_copy(t, dblk, b):
        j = t // i32(_TASKS_PER_TAB)
        i0 = (t % i32(_TASKS_PER_TAB)) * i32(_TB)
        kk = j % i32(_NUM_HEADS)
        for nn in range(3):
            @pl.when(j // i32(_NUM_HEADS) == i32(nn))
            def _(nn=nn, kk=kk, i0=i0, b=b, dblk=dblk):
                pltpu.make_async_copy(
                    srcs[nn].at[pl.ds(kk * i32(_EMBED_DIM) + i32(dblk * 16), 16),
                                pl.ds(i0, _TB)],
                    tins[b], tsems[b]).start()

    def t_in_wait(b):
        pltpu.make_async_copy(s2_hbm.at[pl.ds(i32(0), 16), pl.ds(i32(0), _TB)],
                              tins[b], tsems[b]).wait()

    def t_out_copy(t):
        j = t // i32(_TASKS_PER_TAB)
        i0 = (t % i32(_TASKS_PER_TAB)) * i32(_TB)
        row0 = j * i32(_TABLE_SIZE) + i0
        pltpu.make_async_copy(tout_v, rm_hbm.at[pl.ds(row0, _TB)], sem2).start()

    def t_out_wait():
        pltpu.make_async_copy(tout_v, rm_hbm.at[pl.ds(i32(0), _TB)],
                              sem2).wait()

    def t_shuffle(dblk, b):
        col0 = i32(dblk * 16)

        def srow(li4, carry, b=b, col0=col0):
            for u in range(4):
                li = li4 * i32(4) + i32(u)
                v = plsc.load_gather(tins[b], [iota16, iota16 * i32(0) + li])
                tout_v[li, pl.ds(col0, 16)] = v
            return carry

        lax.fori_loop(jnp.int32(0), jnp.int32(_TB // 4), srow, jnp.int32(0))

    t_in_copy(w, 0, 0)

    def tstep(m, carry):
        t = w + m * i32(_NW)
        tnext = w + (m + i32(1)) * i32(_NW)

        @pl.when(t < i32(_NTASK))
        def _(t=t, tnext=tnext, m=m):
            for dblk in range(_NDB):
                b = dblk % 2
                t_in_wait(b)
                if dblk + 1 < _NDB:
                    t_in_copy(t, dblk + 1, 1 - b)
                else:
                    @pl.when(tnext < i32(_NTASK))
                    def _(tnext=tnext, b=b):
                        t_in_copy(tnext, 0, 1 - b)
                if dblk == 0:
                    @pl.when(m > i32(0))
                    def _():
                        t_out_wait()
                t_shuffle(dblk, b)
            t_out_copy(t)
        return carry

    lax.fori_loop(jnp.int32(0), jnp.int32(_MSTEPS), tstep, jnp.int32(0))
    t_out_wait()

    flg_v[i32(0), :] = iota16 * i32(0) + i32(1)
    pltpu.sync_copy(flg_v.at[i32(0)], flags_hbm.at[w])

    def bcond(carry):
        return carry < i32(_NW * 16)

    def bstep(carry):
        pltpu.sync_copy(flags_hbm, flg_v)
        acc = flg_v[i32(0), :]
        for r in range(1, _NW):
            acc = acc + flg_v[i32(r), :]
        return lax.reduce_sum_p.bind(acc, axes=(0,))

    lax.while_loop(bcond, bstep, jnp.int32(0))

    bufs = (buf0_v, buf1_v)
    gsems = (sem0, sem1)
    wsems = (sem2, sem3)

    def fire_gather(j, c, b):
        pltpu.async_copy(rm_hbm.at[idx_v.at[jnp.int32(j), c]], bufs[b],
                         gsems[b])

    def drain_gather(j, c, b):
        pltpu.make_async_copy(rm_hbm.at[idx_v.at[jnp.int32(j), c]], bufs[b],
                              gsems[b]).wait()

    def fire_write(j, c, b):
        pltpu.make_async_copy(
            bufs[b],
            out_hbm.at[pl.ds(base_row + c * jnp.int32(_CH), _CH), jnp.int32(j)],
            wsems[b]).start()

    def drain_write(j, c, b):
        pltpu.make_async_copy(
            bufs[b],
            out_hbm.at[pl.ds(base_row + c * jnp.int32(_CH), _CH), jnp.int32(j)],
            wsems[b]).wait()

    fire_gather(0, jnp.int32(0), 0)
    fire_gather(0, jnp.int32(1), 1)
    for j in range(_NJ):

        def pstep(c2, carry, j=j):
            for b in range(2):
                c = c2 * i32(2) + i32(b)
                drain_gather(j, c, b)
                fire_write(j, c, b)
            for b in range(2):
                c = c2 * i32(2) + i32(b)
                drain_write(j, c, b)

                @pl.when(c2 < i32(_NCH // 2 - 1))
                def _(j=j, c=c, b=b):
                    fire_gather(j, c + i32(2), b)
            return carry

        lax.fori_loop(jnp.int32(0), jnp.int32(_NCH // 2), pstep, jnp.int32(0))
        if j + 1 < _NJ:
            fire_gather(j + 1, jnp.int32(0), 0)
            fire_gather(j + 1, jnp.int32(1), 1)


def kernel(ngrams_2, ngrams_3, ngrams_4, tables_2, tables_3, tables_4):
    ng2 = ngrams_2.reshape(_BS, 2).astype(jnp.int32)
    ng3 = ngrams_3.reshape(_BS, 3).astype(jnp.int32)
    ng4 = ngrams_4.reshape(_BS, 4).astype(jnp.int32)
    ngall = jnp.concatenate([ng2, ng3, ng4], axis=1)
    ngq = ngall.reshape(_B, _S, 9).transpose(1, 0, 2).reshape(_BS, 9)
    ngt = ngq.reshape(_NW, _C, 9).transpose(0, 2, 1)

    s2 = tables_2.transpose(0, 2, 1).reshape(_NUM_HEADS * _EMBED_DIM, _TABLE_SIZE)
    s3 = tables_3.transpose(0, 2, 1).reshape(_NUM_HEADS * _EMBED_DIM, _TABLE_SIZE)
    s4 = tables_4.transpose(0, 2, 1).reshape(_NUM_HEADS * _EMBED_DIM, _TABLE_SIZE)

    mesh = plsc.VectorSubcoreMesh(core_axis_name="c", subcore_axis_name="s",
                                  num_cores=_NC, num_subcores=_NS)
    run = pl.kernel(
        _body,
        out_type=jax.ShapeDtypeStruct((_BS, _NJ, _EMBED_DIM), jnp.float32),
        mesh=mesh,
        scratch_types=[
            pltpu.HBM((_NJ * _TABLE_SIZE, _EMBED_DIM), jnp.float32),
            pltpu.HBM((_NW, 16), jnp.int32),
            pltpu.VMEM((9, _C), jnp.int32),
            pltpu.VMEM((_NJ, _NCH, _CH), jnp.int32),
            pltpu.VMEM((16, _TB), jnp.float32),
            pltpu.VMEM((16, _TB), jnp.float32),
            pltpu.VMEM((_TB, _EMBED_DIM), jnp.float32),
            pltpu.VMEM((_CH, _EMBED_DIM), jnp.float32),
            pltpu.VMEM((_CH, _EMBED_DIM), jnp.float32),
            pltpu.VMEM((_NW, 16), jnp.int32),
            pltpu.SemaphoreType.DMA,
            pltpu.SemaphoreType.DMA,
            pltpu.SemaphoreType.DMA,
            pltpu.SemaphoreType.DMA,
        ],
        compiler_params=pltpu.CompilerParams(use_tc_tiling_on_sc=False,
                                             needs_layout_passes=False),
    )
    out = run(ngt, s2, s3, s4)
    out = out.reshape(_S, _B, _NJ * _EMBED_DIM)
    return out.transpose(1, 0, 2)


# speedup vs baseline: 1.5577x; 1.5577x over previous
import jax
import jax.numpy as jnp
from jax import lax
from jax.experimental import pallas as pl
from jax.experimental.pallas import tpu as pltpu
from jax.experimental.pallas import tpu_sc as plsc

_MIN_N, _MAX_N = 2, 4
_NUM_HEADS = 4
_TABLE_SIZE = 100000
_EMBED_DIM = 64
_B, _S = 1024, 50
_BS = _B * _S


def _get_prime(n):
    def is_prime(x):
        if x < 2:
            return False
        for i in range(2, int(x ** 0.5) + 1):
            if x % i == 0:
                return False
        return True
    while not is_prime(n):
        n += 1
    return n


_BASES = [_get_prime(i * 100 + 31) for i in range(_NUM_HEADS)]
_MODULI = [_get_prime(_TABLE_SIZE + i * 1000) for i in range(_NUM_HEADS)]

_C0 = {n: [[pow(_BASES[k], i, _MODULI[k]) for i in range(n)]
           for k in range(_NUM_HEADS)] for n in range(_MIN_N, _MAX_N + 1)}
_C1 = {n: [[(256 * pow(_BASES[k], i, _MODULI[k])) % _MODULI[k] for i in range(n)]
           for k in range(_NUM_HEADS)] for n in range(_MIN_N, _MAX_N + 1)}
_ROW0 = {2: 0, 3: 2, 4: 5}

_NC, _NS = 2, 16
_NW = _NC * _NS
_C = _BS // _NW
_CH = 80
_NCH = _C // _CH
_NJ = (_MAX_N - _MIN_N + 1) * _NUM_HEADS

_TB = 800
_TASKS_PER_TAB = _TABLE_SIZE // _TB
_NTASK = _NJ * _TASKS_PER_TAB
_MSTEPS = -(-_NTASK // _NW)
_NDB = _EMBED_DIM // 16


def _body(ng_hbm, s2_hbm, s3_hbm, s4_hbm, out_hbm,
          rm_hbm, flags_hbm,
          ng_v, idx_v, tin0_v, tin1_v, tout_v, buf0_v, buf1_v,
          flg_v, sem0, sem1, sem2, sem3):
    i32 = jnp.int32
    w = lax.axis_index("s") * i32(_NC) + lax.axis_index("c")
    base_row = w * i32(_C)
    iota16 = lax.iota(jnp.int32, 16)

    flg_v[i32(0), :] = iota16 * i32(0)
    pltpu.sync_copy(flg_v.at[i32(0)], flags_hbm.at[w])

    pltpu.sync_copy(ng_hbm.at[w], ng_v)

    def hstep(c, carry):
        for t5 in range(_CH // 16):
            off = c * i32(_CH) + i32(t5 * 16)
            for n in range(_MIN_N, _MAX_N + 1):
                r0 = _ROW0[n]
                gs = [ng_v[r0 + i, pl.ds(off, 16)] for i in range(n)]
                g1 = [g >> 8 for g in gs]
                g0 = [g & 255 for g in gs]
                for k in range(_NUM_HEADS):
                    j = (n - _MIN_N) * _NUM_HEADS + k
                    acc = g1[0] * _C1[n][k][0] + g0[0] * _C0[n][k][0]
                    for i in range(1, n):
                        acc = acc + g1[i] * _C1[n][k][i] + g0[i] * _C0[n][k][i]
                    h = acc % _MODULI[k]
                    h = jnp.where(h >= _TABLE_SIZE, h - _TABLE_SIZE, h)
                    idx_v[j, c, pl.ds(t5 * 16, 16)] = h + j * _TABLE_SIZE
        return carry

    lax.fori_loop(jnp.int32(0), jnp.int32(_NCH), hstep, jnp.int32(0))

    srcs = (s2_hbm, s3_hbm, s4_hbm)
    tins = (tin0_v, tin1_v)
    tsems = (sem0, sem1)

    def t_in_copy(t, dblk, b):
        j = t // i32(_TASKS_PER_TAB)
        i0 = (t % i32(_TASKS_PER_TAB)) * i32(_TB)
        kk = j % i32(_NUM_HEADS)
        for nn in range(3):
            @pl.when(j // i32(_NUM_HEADS) == i32(nn))
            def _(nn=nn, kk=kk, i0=i0, b=b, dblk=dblk):
                pltpu.make_async_copy(
                    srcs[nn].at[pl.ds(kk * i32(_EMBED_DIM) + i32(dblk * 16), 16),
                                pl.ds(i0, _TB)],
                    tins[b].at[:, pl.ds(i32(0), _TB)], tsems[b]).start()

    def t_in_wait(b):
        pltpu.make_async_copy(s2_hbm.at[pl.ds(i32(0), 16), pl.ds(i32(0), _TB)],
                              tins[b].at[:, pl.ds(i32(0), _TB)],
                              tsems[b]).wait()

    def t_out_copy(t):
        j = t // i32(_TASKS_PER_TAB)
        i0 = (t % i32(_TASKS_PER_TAB)) * i32(_TB)
        row0 = j * i32(_TABLE_SIZE) + i0
        pltpu.make_async_copy(tout_v, rm_hbm.at[pl.ds(row0, _TB)], sem2).start()

    def t_out_wait():
        pltpu.make_async_copy(tout_v, rm_hbm.at[pl.ds(i32(0), _TB)],
                              sem2).wait()

    def t_shuffle(dblk, b):
        col0 = i32(dblk * 16)

        def srow(li4, carry, b=b, col0=col0):
            for u in range(4):
                li = li4 * i32(4) + i32(u)
                v = plsc.load_gather(tins[b], [iota16, iota16 * i32(0) + li])
                tout_v[li, pl.ds(col0, 16)] = v
            return carry

        lax.fori_loop(jnp.int32(0), jnp.int32(_TB // 4), srow, jnp.int32(0))

    t_in_copy(w, 0, 0)

    def tstep(m, carry):
        t = w + m * i32(_NW)
        tnext = w + (m + i32(1)) * i32(_NW)

        @pl.when(t < i32(_NTASK))
        def _(t=t, tnext=tnext, m=m):
            for dblk in range(_NDB):
                b = dblk % 2
                t_in_wait(b)
                if dblk + 1 < _NDB:
                    t_in_copy(t, dblk + 1, 1 - b)
                else:
                    @pl.when(tnext < i32(_NTASK))
                    def _(tnext=tnext, b=b):
                        t_in_copy(tnext, 0, 1 - b)
                if dblk == 0:
                    @pl.when(m > i32(0))
                    def _():
                        t_out_wait()
                t_shuffle(dblk, b)
            t_out_copy(t)
        return carry

    lax.fori_loop(jnp.int32(0), jnp.int32(_MSTEPS), tstep, jnp.int32(0))
    t_out_wait()

    flg_v[i32(0), :] = iota16 * i32(0) + i32(1)
    pltpu.sync_copy(flg_v.at[i32(0)], flags_hbm.at[w])

    def bcond(carry):
        return carry < i32(_NW * 16)

    def bstep(carry):
        pltpu.sync_copy(flags_hbm, flg_v)
        acc = flg_v[i32(0), :]
        for r in range(1, _NW):
            acc = acc + flg_v[i32(r), :]
        return lax.reduce_sum_p.bind(acc, axes=(0,))

    lax.while_loop(bcond, bstep, jnp.int32(0))

    bufs = (buf0_v, buf1_v)
    gsems = (sem0, sem1)
    wsems = (sem2, sem3)

    def fire_gather(j, c, b):
        pltpu.async_copy(rm_hbm.at[idx_v.at[jnp.int32(j), c]], bufs[b],
                         gsems[b])

    def drain_gather(j, c, b):
        pltpu.make_async_copy(rm_hbm.at[idx_v.at[jnp.int32(j), c]], bufs[b],
                              gsems[b]).wait()

    def fire_write(j, c, b):
        pltpu.make_async_copy(
            bufs[b],
            out_hbm.at[pl.ds(base_row + c * jnp.int32(_CH), _CH), jnp.int32(j)],
            wsems[b]).start()

    def drain_write(j, c, b):
        pltpu.make_async_copy(
            bufs[b],
            out_hbm.at[pl.ds(base_row + c * jnp.int32(_CH), _CH), jnp.int32(j)],
            wsems[b]).wait()

    fire_gather(0, jnp.int32(0), 0)
    fire_gather(0, jnp.int32(1), 1)
    for j in range(_NJ):

        def pstep(c2, carry, j=j):
            for b in range(2):
                c = c2 * i32(2) + i32(b)
                drain_gather(j, c, b)
                fire_write(j, c, b)
            for b in range(2):
                c = c2 * i32(2) + i32(b)
                drain_write(j, c, b)

                @pl.when(c2 < i32(_NCH // 2 - 1))
                def _(j=j, c=c, b=b):
                    fire_gather(j, c + i32(2), b)
            return carry

        lax.fori_loop(jnp.int32(0), jnp.int32(_NCH // 2), pstep, jnp.int32(0))
        if j + 1 < _NJ:
            fire_gather(j + 1, jnp.int32(0), 0)
            fire_gather(j + 1, jnp.int32(1), 1)


def kernel(ngrams_2, ngrams_3, ngrams_4, tables_2, tables_3, tables_4):
    ng2 = ngrams_2.reshape(_BS, 2).astype(jnp.int32)
    ng3 = ngrams_3.reshape(_BS, 3).astype(jnp.int32)
    ng4 = ngrams_4.reshape(_BS, 4).astype(jnp.int32)
    ngall = jnp.concatenate([ng2, ng3, ng4], axis=1)
    ngq = ngall.reshape(_B, _S, 9).transpose(1, 0, 2).reshape(_BS, 9)
    ngt = ngq.reshape(_NW, _C, 9).transpose(0, 2, 1)

    s2 = tables_2.transpose(0, 2, 1).reshape(_NUM_HEADS * _EMBED_DIM, _TABLE_SIZE)
    s3 = tables_3.transpose(0, 2, 1).reshape(_NUM_HEADS * _EMBED_DIM, _TABLE_SIZE)
    s4 = tables_4.transpose(0, 2, 1).reshape(_NUM_HEADS * _EMBED_DIM, _TABLE_SIZE)

    mesh = plsc.VectorSubcoreMesh(core_axis_name="c", subcore_axis_name="s",
                                  num_cores=_NC, num_subcores=_NS)
    run = pl.kernel(
        _body,
        out_type=jax.ShapeDtypeStruct((_BS, _NJ, _EMBED_DIM), jnp.float32),
        mesh=mesh,
        scratch_types=[
            pltpu.HBM((_NJ * _TABLE_SIZE, _EMBED_DIM), jnp.float32),
            pltpu.HBM((_NW, 16), jnp.int32),
            pltpu.VMEM((9, _C), jnp.int32),
            pltpu.VMEM((_NJ, _NCH, _CH), jnp.int32),
            pltpu.VMEM((16, _TB + 1), jnp.float32),
            pltpu.VMEM((16, _TB + 1), jnp.float32),
            pltpu.VMEM((_TB, _EMBED_DIM), jnp.float32),
            pltpu.VMEM((_CH, _EMBED_DIM), jnp.float32),
            pltpu.VMEM((_CH, _EMBED_DIM), jnp.float32),
            pltpu.VMEM((_NW, 16), jnp.int32),
            pltpu.SemaphoreType.DMA,
            pltpu.SemaphoreType.DMA,
            pltpu.SemaphoreType.DMA,
            pltpu.SemaphoreType.DMA,
        ],
        compiler_params=pltpu.CompilerParams(use_tc_tiling_on_sc=False,
                                             needs_layout_passes=False),
    )
    out = run(ngt, s2, s3, s4)
    out = out.reshape(_S, _B, _NJ * _EMBED_DIM)
    return out.transpose(1, 0, 2)


# speedup vs baseline: 2.3865x; 1.5321x over previous
import jax
import jax.numpy as jnp
from jax import lax
from jax.experimental import pallas as pl
from jax.experimental.pallas import tpu as pltpu
from jax.experimental.pallas import tpu_sc as plsc

_MIN_N, _MAX_N = 2, 4
_NUM_HEADS = 4
_TABLE_SIZE = 100000
_EMBED_DIM = 64
_B, _S = 1024, 50
_BS = _B * _S


def _get_prime(n):
    def is_prime(x):
        if x < 2:
            return False
        for i in range(2, int(x ** 0.5) + 1):
            if x % i == 0:
                return False
        return True
    while not is_prime(n):
        n += 1
    return n


_BASES = [_get_prime(i * 100 + 31) for i in range(_NUM_HEADS)]
_MODULI = [_get_prime(_TABLE_SIZE + i * 1000) for i in range(_NUM_HEADS)]

_C0 = {n: [[pow(_BASES[k], i, _MODULI[k]) for i in range(n)]
           for k in range(_NUM_HEADS)] for n in range(_MIN_N, _MAX_N + 1)}
_C1 = {n: [[(256 * pow(_BASES[k], i, _MODULI[k])) % _MODULI[k] for i in range(n)]
           for k in range(_NUM_HEADS)] for n in range(_MIN_N, _MAX_N + 1)}
_ROW0 = {2: 0, 3: 2, 4: 5}

_NC, _NS = 2, 16
_NW = _NC * _NS
_C = _BS // _NW
_CH = 80
_NCH = _C // _CH
_NJ = (_MAX_N - _MIN_N + 1) * _NUM_HEADS

_TB = 800
_TASKS_PER_TAB = _TABLE_SIZE // _TB
_NTASK = _NJ * _TASKS_PER_TAB
_MSTEPS = -(-_NTASK // _NW)
_NDB = _EMBED_DIM // 16


def _body(ng_hbm, s2_hbm, s3_hbm, s4_hbm, out_hbm,
          rm_hbm, flags_hbm,
          ng_v, idx_v, tin0_v, tin1_v, tout_v, buf0_v, buf1_v,
          flg_v, sem0, sem1, sem2, sem3):
    i32 = jnp.int32
    w = lax.axis_index("s") * i32(_NC) + lax.axis_index("c")
    base_row = w * i32(_C)
    iota16 = lax.iota(jnp.int32, 16)

    flg_v[i32(0), :] = iota16 * i32(0)
    pltpu.sync_copy(flg_v.at[i32(0)], flags_hbm.at[w])

    pltpu.sync_copy(ng_hbm.at[w], ng_v)

    def hstep(c, carry):
        for t5 in range(_CH // 16):
            off = c * i32(_CH) + i32(t5 * 16)
            for n in range(_MIN_N, _MAX_N + 1):
                r0 = _ROW0[n]
                gs = [ng_v[r0 + i, pl.ds(off, 16)] for i in range(n)]
                g1 = [g >> 8 for g in gs]
                g0 = [g & 255 for g in gs]
                for k in range(_NUM_HEADS):
                    j = (n - _MIN_N) * _NUM_HEADS + k
                    acc = g1[0] * _C1[n][k][0] + g0[0] * _C0[n][k][0]
                    for i in range(1, n):
                        acc = acc + g1[i] * _C1[n][k][i] + g0[i] * _C0[n][k][i]
                    h = acc % _MODULI[k]
                    h = jnp.where(h >= _TABLE_SIZE, h - _TABLE_SIZE, h)
                    idx_v[j, c, pl.ds(t5 * 16, 16)] = h + j * _TABLE_SIZE
        return carry

    lax.fori_loop(jnp.int32(0), jnp.int32(_NCH), hstep, jnp.int32(0))

    srcs = (s2_hbm, s3_hbm, s4_hbm)
    tins = (tin0_v, tin1_v)
    tsems = (sem0, sem1)

    def t_in_copy(t, dblk, b):
        j = t // i32(_TASKS_PER_TAB)
        i0 = (t % i32(_TASKS_PER_TAB)) * i32(_TB)
        kk = j % i32(_NUM_HEADS)
        for nn in range(3):
            @pl.when(j // i32(_NUM_HEADS) == i32(nn))
            def _(nn=nn, kk=kk, i0=i0, b=b, dblk=dblk):
                pltpu.make_async_copy(
                    srcs[nn].at[pl.ds(kk * i32(_EMBED_DIM) + i32(dblk * 16), 16),
                                pl.ds(i0, _TB)],
                    tins[b].at[:, pl.ds(i32(0), _TB)], tsems[b]).start()

    def t_in_wait(b):
        pltpu.make_async_copy(s2_hbm.at[pl.ds(i32(0), 16), pl.ds(i32(0), _TB)],
                              tins[b].at[:, pl.ds(i32(0), _TB)],
                              tsems[b]).wait()

    def t_out_copy(t):
        j = t // i32(_TASKS_PER_TAB)
        i0 = (t % i32(_TASKS_PER_TAB)) * i32(_TB)
        row0 = j * i32(_TABLE_SIZE) + i0
        pltpu.make_async_copy(tout_v, rm_hbm.at[pl.ds(row0, _TB)], sem2).start()

    def t_out_wait():
        pltpu.make_async_copy(tout_v, rm_hbm.at[pl.ds(i32(0), _TB)],
                              sem2).wait()

    def t_shuffle(dblk, b):
        col0 = i32(dblk * 16)

        tout_v[i32(0), pl.ds(col0, 16)] = tins[b][i32(0), pl.ds(0, 16)]

    t_in_copy(w, 0, 0)

    def tstep(m, carry):
        t = w + m * i32(_NW)
        tnext = w + (m + i32(1)) * i32(_NW)

        @pl.when(t < i32(_NTASK))
        def _(t=t, tnext=tnext, m=m):
            for dblk in range(_NDB):
                b = dblk % 2
                t_in_wait(b)
                if dblk + 1 < _NDB:
                    t_in_copy(t, dblk + 1, 1 - b)
                else:
                    @pl.when(tnext < i32(_NTASK))
                    def _(tnext=tnext, b=b):
                        t_in_copy(tnext, 0, 1 - b)
                if dblk == 0:
                    @pl.when(m > i32(0))
                    def _():
                        t_out_wait()
                t_shuffle(dblk, b)
            t_out_copy(t)
        return carry

    lax.fori_loop(jnp.int32(0), jnp.int32(_MSTEPS), tstep, jnp.int32(0))
    t_out_wait()

    flg_v[i32(0), :] = iota16 * i32(0) + i32(1)
    pltpu.sync_copy(flg_v.at[i32(0)], flags_hbm.at[w])

    def bcond(carry):
        return carry < i32(_NW * 16)

    def bstep(carry):
        pltpu.sync_copy(flags_hbm, flg_v)
        acc = flg_v[i32(0), :]
        for r in range(1, _NW):
            acc = acc + flg_v[i32(r), :]
        return lax.reduce_sum_p.bind(acc, axes=(0,))

    lax.while_loop(bcond, bstep, jnp.int32(0))

    bufs = (buf0_v, buf1_v)
    gsems = (sem0, sem1)
    wsems = (sem2, sem3)

    def fire_gather(j, c, b):
        pltpu.async_copy(rm_hbm.at[idx_v.at[jnp.int32(j), c]], bufs[b],
                         gsems[b])

    def drain_gather(j, c, b):
        pltpu.make_async_copy(rm_hbm.at[idx_v.at[jnp.int32(j), c]], bufs[b],
                              gsems[b]).wait()

    def fire_write(j, c, b):
        pltpu.make_async_copy(
            bufs[b],
            out_hbm.at[pl.ds(base_row + c * jnp.int32(_CH), _CH), jnp.int32(j)],
            wsems[b]).start()

    def drain_write(j, c, b):
        pltpu.make_async_copy(
            bufs[b],
            out_hbm.at[pl.ds(base_row + c * jnp.int32(_CH), _CH), jnp.int32(j)],
            wsems[b]).wait()

    fire_gather(0, jnp.int32(0), 0)
    fire_gather(0, jnp.int32(1), 1)
    for j in range(_NJ):

        def pstep(c2, carry, j=j):
            for b in range(2):
                c = c2 * i32(2) + i32(b)
                drain_gather(j, c, b)
                fire_write(j, c, b)
            for b in range(2):
                c = c2 * i32(2) + i32(b)
                drain_write(j, c, b)

                @pl.when(c2 < i32(_NCH // 2 - 1))
                def _(j=j, c=c, b=b):
                    fire_gather(j, c + i32(2), b)
            return carry

        lax.fori_loop(jnp.int32(0), jnp.int32(_NCH // 2), pstep, jnp.int32(0))
        if j + 1 < _NJ:
            fire_gather(j + 1, jnp.int32(0), 0)
            fire_gather(j + 1, jnp.int32(1), 1)


def kernel(ngrams_2, ngrams_3, ngrams_4, tables_2, tables_3, tables_4):
    ng2 = ngrams_2.reshape(_BS, 2).astype(jnp.int32)
    ng3 = ngrams_3.reshape(_BS, 3).astype(jnp.int32)
    ng4 = ngrams_4.reshape(_BS, 4).astype(jnp.int32)
    ngall = jnp.concatenate([ng2, ng3, ng4], axis=1)
    ngq = ngall.reshape(_B, _S, 9).transpose(1, 0, 2).reshape(_BS, 9)
    ngt = ngq.reshape(_NW, _C, 9).transpose(0, 2, 1)

    s2 = tables_2.transpose(0, 2, 1).reshape(_NUM_HEADS * _EMBED_DIM, _TABLE_SIZE)
    s3 = tables_3.transpose(0, 2, 1).reshape(_NUM_HEADS * _EMBED_DIM, _TABLE_SIZE)
    s4 = tables_4.transpose(0, 2, 1).reshape(_NUM_HEADS * _EMBED_DIM, _TABLE_SIZE)

    mesh = plsc.VectorSubcoreMesh(core_axis_name="c", subcore_axis_name="s",
                                  num_cores=_NC, num_subcores=_NS)
    run = pl.kernel(
        _body,
        out_type=jax.ShapeDtypeStruct((_BS, _NJ, _EMBED_DIM), jnp.float32),
        mesh=mesh,
        scratch_types=[
            pltpu.HBM((_NJ * _TABLE_SIZE, _EMBED_DIM), jnp.float32),
            pltpu.HBM((_NW, 16), jnp.int32),
            pltpu.VMEM((9, _C), jnp.int32),
            pltpu.VMEM((_NJ, _NCH, _CH), jnp.int32),
            pltpu.VMEM((16, _TB + 1), jnp.float32),
            pltpu.VMEM((16, _TB + 1), jnp.float32),
            pltpu.VMEM((_TB, _EMBED_DIM), jnp.float32),
            pltpu.VMEM((_CH, _EMBED_DIM), jnp.float32),
            pltpu.VMEM((_CH, _EMBED_DIM), jnp.float32),
            pltpu.VMEM((_NW, 16), jnp.int32),
            pltpu.SemaphoreType.DMA,
            pltpu.SemaphoreType.DMA,
            pltpu.SemaphoreType.DMA,
            pltpu.SemaphoreType.DMA,
        ],
        compiler_params=pltpu.CompilerParams(use_tc_tiling_on_sc=False,
                                             needs_layout_passes=False),
    )
    out = run(ngt, s2, s3, s4)
    out = out.reshape(_S, _B, _NJ * _EMBED_DIM)
    return out.transpose(1, 0, 2)


# speedup vs baseline: 2.6669x; 1.1175x over previous
import jax
import jax.numpy as jnp
from jax import lax
from jax.experimental import pallas as pl
from jax.experimental.pallas import tpu as pltpu
from jax.experimental.pallas import tpu_sc as plsc

_MIN_N, _MAX_N = 2, 4
_NUM_HEADS = 4
_TABLE_SIZE = 100000
_EMBED_DIM = 64
_B, _S = 1024, 50
_BS = _B * _S


def _get_prime(n):
    def is_prime(x):
        if x < 2:
            return False
        for i in range(2, int(x ** 0.5) + 1):
            if x % i == 0:
                return False
        return True
    while not is_prime(n):
        n += 1
    return n


_BASES = [_get_prime(i * 100 + 31) for i in range(_NUM_HEADS)]
_MODULI = [_get_prime(_TABLE_SIZE + i * 1000) for i in range(_NUM_HEADS)]

_C0 = {n: [[pow(_BASES[k], i, _MODULI[k]) for i in range(n)]
           for k in range(_NUM_HEADS)] for n in range(_MIN_N, _MAX_N + 1)}
_C1 = {n: [[(256 * pow(_BASES[k], i, _MODULI[k])) % _MODULI[k] for i in range(n)]
           for k in range(_NUM_HEADS)] for n in range(_MIN_N, _MAX_N + 1)}
_ROW0 = {2: 0, 3: 2, 4: 5}

_NC, _NS = 2, 16
_NW = _NC * _NS
_C = _BS // _NW
_CH = 80
_NCH = _C // _CH
_NJ = (_MAX_N - _MIN_N + 1) * _NUM_HEADS


def _body(ng_hbm, t2_hbm, t3_hbm, t4_hbm, out_hbm, ng_v, idx_v,
          buf0_v, buf1_v, buf2_v, buf3_v,
          gsem0, gsem1, gsem2, gsem3, wsem0, wsem1, wsem2, wsem3):
    i32 = jnp.int32
    w = lax.axis_index("s") * i32(_NC) + lax.axis_index("c")
    base_row = w * i32(_C)
    bufs = (buf0_v, buf1_v, buf2_v, buf3_v)
    gsems = (gsem0, gsem1, gsem2, gsem3)
    wsems = (wsem0, wsem1, wsem2, wsem3)

    pltpu.sync_copy(ng_hbm.at[w], ng_v)

    def hstep(c, carry):
        for t5 in range(_CH // 16):
            off = c * i32(_CH) + i32(t5 * 16)
            for n in range(_MIN_N, _MAX_N + 1):
                r0 = _ROW0[n]
                gs = [ng_v[r0 + i, pl.ds(off, 16)] for i in range(n)]
                g1 = [g >> 8 for g in gs]
                g0 = [g & 255 for g in gs]
                for k in range(_NUM_HEADS):
                    j = (n - _MIN_N) * _NUM_HEADS + k
                    acc = g1[0] * _C1[n][k][0] + g0[0] * _C0[n][k][0]
                    for i in range(1, n):
                        acc = acc + g1[i] * _C1[n][k][i] + g0[i] * _C0[n][k][i]
                    h = acc % _MODULI[k]
                    h = jnp.where(h >= _TABLE_SIZE, h - _TABLE_SIZE, h)
                    idx_v[j, c, pl.ds(t5 * 16, 16)] = h + k * _TABLE_SIZE
        return carry

    lax.fori_loop(jnp.int32(0), jnp.int32(_NCH), hstep, jnp.int32(0))

    tabs = (t2_hbm, t3_hbm, t4_hbm)

    def fire_gather(j, c, b):
        tab = tabs[j // _NUM_HEADS]
        pltpu.async_copy(tab.at[idx_v.at[jnp.int32(j), c]], bufs[b], gsems[b])

    def fire_write(j, c, b):
        pltpu.make_async_copy(
            bufs[b],
            out_hbm.at[pl.ds(base_row + c * jnp.int32(_CH), _CH), jnp.int32(j)],
            wsems[b]).start()

    def drain_write(j, c, b):
        pltpu.make_async_copy(
            bufs[b],
            out_hbm.at[pl.ds(base_row + c * jnp.int32(_CH), _CH), jnp.int32(j)],
            wsems[b]).wait()

    def drain_gather(j, c, b):
        tab = tabs[j // _NUM_HEADS]
        pltpu.make_async_copy(tab.at[idx_v.at[jnp.int32(j), c]], bufs[b],
                              gsems[b]).wait()

    _NB = 4
    for b in range(_NB):
        fire_gather(0, jnp.int32(b), b)
    for j in range(_NJ):

        def pstep(c4, carry, j=j):
            for b in range(_NB):
                c = c4 * i32(_NB) + i32(b)
                drain_gather(j, c, b)
                fire_write(j, c, b)
            for b in range(_NB):
                c = c4 * i32(_NB) + i32(b)
                drain_write(j, c, b)

                @pl.when(c4 < i32(_NCH // _NB - 1))
                def _(j=j, c=c, b=b):
                    fire_gather(j, c + i32(_NB), b)
            return carry

        lax.fori_loop(jnp.int32(0), jnp.int32(_NCH // _NB), pstep, jnp.int32(0))
        if j + 1 < _NJ:
            for b in range(_NB):
                fire_gather(j + 1, jnp.int32(b), b)


def kernel(ngrams_2, ngrams_3, ngrams_4, tables_2, tables_3, tables_4):
    ng2 = ngrams_2.reshape(_BS, 2).astype(jnp.int32)
    ng3 = ngrams_3.reshape(_BS, 3).astype(jnp.int32)
    ng4 = ngrams_4.reshape(_BS, 4).astype(jnp.int32)
    ngall = jnp.concatenate([ng2, ng3, ng4], axis=1)
    ngq = ngall.reshape(_B, _S, 9).transpose(1, 0, 2).reshape(_BS, 9)
    ngt = ngq.reshape(_NW, _C, 9).transpose(0, 2, 1)

    t2 = tables_2.reshape(_NUM_HEADS * _TABLE_SIZE, _EMBED_DIM)
    t3 = tables_3.reshape(_NUM_HEADS * _TABLE_SIZE, _EMBED_DIM)
    t4 = tables_4.reshape(_NUM_HEADS * _TABLE_SIZE, _EMBED_DIM)

    mesh = plsc.VectorSubcoreMesh(core_axis_name="c", subcore_axis_name="s",
                                  num_cores=_NC, num_subcores=_NS)
    run = pl.kernel(
        _body,
        out_type=jax.ShapeDtypeStruct((_BS, _NJ, _EMBED_DIM), jnp.float32),
        mesh=mesh,
        scratch_types=[
            pltpu.VMEM((9, _C), jnp.int32),
            pltpu.VMEM((_NJ, _NCH, _CH), jnp.int32),
            pltpu.VMEM((_CH, _EMBED_DIM), jnp.float32),
            pltpu.VMEM((_CH, _EMBED_DIM), jnp.float32),
            pltpu.VMEM((_CH, _EMBED_DIM), jnp.float32),
            pltpu.VMEM((_CH, _EMBED_DIM), jnp.float32),
            pltpu.SemaphoreType.DMA,
            pltpu.SemaphoreType.DMA,
            pltpu.SemaphoreType.DMA,
            pltpu.SemaphoreType.DMA,
            pltpu.SemaphoreType.DMA,
            pltpu.SemaphoreType.DMA,
            pltpu.SemaphoreType.DMA,
            pltpu.SemaphoreType.DMA,
        ],
        compiler_params=pltpu.CompilerParams(use_tc_tiling_on_sc=False),
    )
    out = run(ngt, t2, t3, t4)
    out = out.reshape(_S, _B, _NJ * _EMBED_DIM)
    return out.transpose(1, 0, 2)
